# baseline (speedup 1.0000x reference)
import functools

import jax
import jax.numpy as jnp
from jax.experimental import pallas as pl
from jax.experimental.pallas import tpu as pltpu

B = 512


def _w_block(pos_a, sq_col, geomT_blk, r2_val, with_diag_mask):
    sq_row = geomT_blk[2:3, :]
    crossm2 = jnp.dot(pos_a, geomT_blk[0:2, :],
                      preferred_element_type=jnp.float32)
    d2 = jnp.maximum((sq_col + sq_row) + crossm2, 1e-12)
    mask = d2 <= r2_val
    if with_diag_mask:
        ri = jax.lax.broadcasted_iota(jnp.int32, (B, B), 0)
        ci = jax.lax.broadcasted_iota(jnp.int32, (B, B), 1)
        mask = mask & (ri != ci)
    return jnp.where(mask, jax.lax.rsqrt(d2), 0.0)


def _sq_col(pos_a):
    px = pos_a[:, 0:1]
    py = pos_a[:, 1:2]
    return px * px + py * py


def _colsum_t(w, m):
    return jax.lax.dot_general(w, m, (((0,), (0,)), ((), ())),
                               preferred_element_type=jnp.float32)


def _sweep_kernel(T, nb, mode, a_ref, b_ref, geomT_ref, pos_a_ref, *rest):
    if mode == "deg":
        x_ref, W1_ref, r2_ref, dinv_ref, M1_ref, acc = rest
    elif mode == "l1":
        M_ref, dinv_ref, Wn_ref, bias_ref, r2_ref, out_ref, acc = rest
    else:
        M_ref, dinv_ref, bias_ref, r2_ref, out_ref, acc = rest

    t = pl.program_id(0)
    a = a_ref[t]
    b = b_ref[t]
    r2 = r2_ref[0:1, 0:1]
    pos_a = pos_a_ref[...]
    sq_col = _sq_col(pos_a)

    @pl.when(t == 0)
    def _init():
        acc[...] = jnp.zeros_like(acc)

    @pl.when(t < nb)
    def _diag():
        w = _w_block(pos_a, sq_col, geomT_ref[...], r2, True)
        if mode == "deg":
            acc[pl.ds(a * B, B), :] += jnp.sum(w, axis=1, keepdims=True)
        else:
            m_a = M_ref[pl.ds(a * B, B), :]
            acc[pl.ds(a * B, B), :] += m_a + jnp.dot(
                w, m_a, preferred_element_type=jnp.float32)

    @pl.when(t >= nb)
    def _off():
        w = _w_block(pos_a, sq_col, geomT_ref[...], r2, False)
        if mode == "deg":
            ones = jnp.ones((B, 1), jnp.float32)
            acc[pl.ds(a * B, B), :] += jnp.sum(w, axis=1, keepdims=True)
            acc[pl.ds(b * B, B), :] += _colsum_t(w, ones)
        else:
            acc[pl.ds(a * B, B), :] += jnp.dot(
                w, M_ref[pl.ds(b * B, B), :],
                preferred_element_type=jnp.float32)
            acc[pl.ds(b * B, B), :] += _colsum_t(w, M_ref[pl.ds(a * B, B), :])

    @pl.when(t == T - 1)
    def _finish():
        if mode == "deg":
            dinv = jax.lax.rsqrt(acc[...] + 1.0)
            dinv_ref[...] = dinv
            xw = jnp.dot(x_ref[...], W1_ref[...],
                         preferred_element_type=jnp.float32)
            M1_ref[...] = dinv * xw
        elif mode == "l1":
            y = dinv_ref[...] * acc[...] + bias_ref[0:1, :]
            h = jax.nn.relu(y)
            out_ref[...] = dinv_ref[...] * jnp.dot(
                h, Wn_ref[...], preferred_element_type=jnp.float32)
        else:
            out_ref[...] = dinv_ref[...] * acc[...] + bias_ref[0:1, :]


def kernel(x, pos, r, W1, b1, W2, b2):
    n, feat = x.shape
    h1 = W1.shape[1]
    h2 = W2.shape[1]
    nb = -(-n // B)
    np_ = nb * B

    pad = np_ - n
    fill = 1e6 + 1e3 * jnp.arange(pad, dtype=jnp.float32)
    pos_p = jnp.concatenate([pos, jnp.stack([fill, fill], axis=1)], axis=0)
    x_p = jnp.concatenate([x, jnp.zeros((pad, feat), x.dtype)], axis=0)
    sq_p = jnp.sum(pos_p * pos_p, axis=1)
    geomT = jnp.concatenate([-2.0 * pos_p.T, sq_p[None, :],
                             jnp.zeros((5, np_), jnp.float32)], axis=0)
    r_f = jnp.asarray(r, jnp.float32)
    r2_b = jnp.full((1, 128), r_f * r_f, jnp.float32)
    b1_2 = b1.reshape(1, h1)
    b2_2 = b2.reshape(1, h2)

    pairs = [(t, t) for t in range(nb)]
    pairs += [(a, b) for a in range(nb) for b in range(a + 1, nb)]
    T = len(pairs)
    a_idx = jnp.array([p[0] for p in pairs], jnp.int32)
    b_idx = jnp.array([p[1] for p in pairs], jnp.int32)

    geom_spec = pl.BlockSpec((8, B), lambda t, ar, br: (0, br[t]))
    pos_spec = pl.BlockSpec((B, 2), lambda t, ar, br: (ar[t], 0))
    full = lambda shape: pl.BlockSpec(shape, lambda t, ar, br: (0, 0))

    dinv, M1 = pl.pallas_call(
        functools.partial(_sweep_kernel, T, nb, "deg"),
        grid_spec=pltpu.PrefetchScalarGridSpec(
            num_scalar_prefetch=2,
            grid=(T,),
            in_specs=[geom_spec, pos_spec, full((np_, feat)),
                      full((feat, h1)), full((1, 128))],
            out_specs=[full((np_, 1)), full((np_, h1))],
            scratch_shapes=[pltpu.VMEM((np_, 1), jnp.float32)],
        ),
        out_shape=[jax.ShapeDtypeStruct((np_, 1), jnp.float32),
                   jax.ShapeDtypeStruct((np_, h1), jnp.float32)],
    )(a_idx, b_idx, geomT, pos_p, x_p, W1, r2_b)

    M2 = pl.pallas_call(
        functools.partial(_sweep_kernel, T, nb, "l1"),
        grid_spec=pltpu.PrefetchScalarGridSpec(
            num_scalar_prefetch=2,
            grid=(T,),
            in_specs=[geom_spec, pos_spec, full((np_, h1)), full((np_, 1)),
                      full((h1, h2)), full((1, h1)), full((1, 128))],
            out_specs=full((np_, h2)),
            scratch_shapes=[pltpu.VMEM((np_, h1), jnp.float32)],
        ),
        out_shape=jax.ShapeDtypeStruct((np_, h2), jnp.float32),
    )(a_idx, b_idx, geomT, pos_p, M1, dinv, W2, b1_2, r2_b)

    out = pl.pallas_call(
        functools.partial(_sweep_kernel, T, nb, "l2"),
        grid_spec=pltpu.PrefetchScalarGridSpec(
            num_scalar_prefetch=2,
            grid=(T,),
            in_specs=[geom_spec, pos_spec, full((np_, h2)), full((np_, 1)),
                      full((1, h2)), full((1, 128))],
            out_specs=full((np_, h2)),
            scratch_shapes=[pltpu.VMEM((np_, h2), jnp.float32)],
        ),
        out_shape=jax.ShapeDtypeStruct((np_, h2), jnp.float32),
    )(a_idx, b_idx, geomT, pos_p, M2, dinv, b2_2, r2_b)

    return out[:n]


# speedup vs baseline: 2.3532x; 2.3532x over previous
import functools

import jax
import jax.numpy as jnp
from jax.experimental import pallas as pl
from jax.experimental.pallas import tpu as pltpu

B = 512


def _w_block(pos_c, sq_col, geomT_ref, r2_val, a, b):
    sq_row = geomT_ref[2:3, b * B:(b + 1) * B]
    crossm2 = jnp.dot(pos_c, geomT_ref[0:2, b * B:(b + 1) * B],
                      preferred_element_type=jnp.float32)
    d2 = jnp.maximum((sq_col + sq_row) + crossm2, 1e-12)
    row_ids = a * B + jax.lax.broadcasted_iota(jnp.int32, (B, B), 0)
    col_ids = b * B + jax.lax.broadcasted_iota(jnp.int32, (B, B), 1)
    mask = (d2 <= r2_val) & (row_ids != col_ids)
    return jnp.where(mask, jax.lax.rsqrt(d2), 0.0)


def _sq_col(pos_c):
    px = pos_c[:, 0:1]
    py = pos_c[:, 1:2]
    return px * px + py * py


def _active(s_ref, nb, a, b):
    r = s_ref[2 * nb]
    return ((s_ref[b] <= s_ref[nb + a] + r)
            & (s_ref[nb + b] >= s_ref[a] - r))


def _deg_kernel(nb, s_ref, geomT_ref, pos_c_ref, x_ref, W1_ref, r2_ref,
                dinv_ref, M1_ref, acc):
    a = pl.program_id(0)
    r2 = r2_ref[0:1, 0:1]
    pos_c = pos_c_ref[...]
    sq_col = _sq_col(pos_c)
    acc[...] = jnp.zeros_like(acc)
    for b in range(nb):
        @pl.when(_active(s_ref, nb, a, b))
        def _blk():
            w = _w_block(pos_c, sq_col, geomT_ref, r2, a, b)
            acc[...] += jnp.sum(w, axis=1, keepdims=True)
    deg = acc[...] + 1.0
    dinv = jax.lax.rsqrt(deg)
    dinv_ref[...] = jnp.broadcast_to(dinv, (B, 8))
    xw = jnp.dot(x_ref[...], W1_ref[...], preferred_element_type=jnp.float32)
    M1_ref[...] = dinv * xw


def _agg_kernel(nb, relu_next, s_ref, geomT_ref, pos_c_ref, M_ref, dinv_ref,
                Wn_ref, bias_ref, r2_ref, out_ref, acc):
    a = pl.program_id(0)
    r2 = r2_ref[0:1, 0:1]
    pos_c = pos_c_ref[...]
    sq_col = _sq_col(pos_c)
    acc[...] = M_ref[pl.ds(a * B, B), :]
    for b in range(nb):
        @pl.when(_active(s_ref, nb, a, b))
        def _blk():
            w = _w_block(pos_c, sq_col, geomT_ref, r2, a, b)
            acc[...] += jnp.dot(w, M_ref[b * B:(b + 1) * B, :],
                                preferred_element_type=jnp.float32)
    dinv = dinv_ref[:, 0:1]
    y = dinv * acc[...] + bias_ref[0:1, :]
    if relu_next:
        h = jax.nn.relu(y)
        out_ref[...] = dinv * jnp.dot(h, Wn_ref[...],
                                      preferred_element_type=jnp.float32)
    else:
        out_ref[...] = y


def kernel(x, pos, r, W1, b1, W2, b2):
    n, feat = x.shape
    h1 = W1.shape[1]
    h2 = W2.shape[1]
    nb = -(-n // B)
    np_ = nb * B

    perm = jnp.argsort(pos[:, 1])
    pos_s = pos[perm]
    x_s = x[perm]

    pad = np_ - n
    fill = 1e6 + 1e3 * jnp.arange(pad, dtype=jnp.float32)
    pos_p = jnp.concatenate([pos_s, jnp.stack([fill, fill], axis=1)], axis=0)
    x_p = jnp.concatenate([x_s, jnp.zeros((pad, feat), x.dtype)], axis=0)
    sq_p = jnp.sum(pos_p * pos_p, axis=1)
    geomT = jnp.concatenate([-2.0 * pos_p.T, sq_p[None, :],
                             jnp.zeros((5, np_), jnp.float32)], axis=0)
    r_f = jnp.asarray(r, jnp.float32)
    r2_b = jnp.full((1, 128), r_f * r_f, jnp.float32)
    b1_2 = b1.reshape(1, h1)
    b2_2 = b2.reshape(1, h2)

    y_p = pos_p[:, 1]
    scal = jnp.concatenate([y_p[0::B], y_p[B - 1::B], r_f[None]])

    full = lambda shape: pl.BlockSpec(shape, lambda a, s: (0, 0))
    rowblk = lambda w: pl.BlockSpec((B, w), lambda a, s: (a, 0))

    dinv, M1 = pl.pallas_call(
        functools.partial(_deg_kernel, nb),
        grid_spec=pltpu.PrefetchScalarGridSpec(
            num_scalar_prefetch=1,
            grid=(nb,),
            in_specs=[full((8, np_)), rowblk(2), rowblk(feat),
                      full((feat, h1)), full((1, 128))],
            out_specs=[rowblk(8), rowblk(h1)],
            scratch_shapes=[pltpu.VMEM((B, 1), jnp.float32)],
        ),
        out_shape=[jax.ShapeDtypeStruct((np_, 8), jnp.float32),
                   jax.ShapeDtypeStruct((np_, h1), jnp.float32)],
    )(scal, geomT, pos_p, x_p, W1, r2_b)

    M2 = pl.pallas_call(
        functools.partial(_agg_kernel, nb, True),
        grid_spec=pltpu.PrefetchScalarGridSpec(
            num_scalar_prefetch=1,
            grid=(nb,),
            in_specs=[full((8, np_)), rowblk(2), full((np_, h1)), rowblk(8),
                      full((h1, h2)), full((1, h1)), full((1, 128))],
            out_specs=rowblk(h2),
            scratch_shapes=[pltpu.VMEM((B, h1), jnp.float32)],
        ),
        out_shape=jax.ShapeDtypeStruct((np_, h2), jnp.float32),
    )(scal, geomT, pos_p, M1, dinv, W2, b1_2, r2_b)

    out_s = pl.pallas_call(
        functools.partial(_agg_kernel, nb, False),
        grid_spec=pltpu.PrefetchScalarGridSpec(
            num_scalar_prefetch=1,
            grid=(nb,),
            in_specs=[full((8, np_)), rowblk(2), full((np_, h2)), rowblk(8),
                      full((h1, h2)), full((1, h2)), full((1, 128))],
            out_specs=rowblk(h2),
            scratch_shapes=[pltpu.VMEM((B, h2), jnp.float32)],
        ),
        out_shape=jax.ShapeDtypeStruct((np_, h2), jnp.float32),
    )(scal, geomT, pos_p, M2, dinv, W2, b2_2, r2_b)

    inv = jnp.zeros((n,), jnp.int32).at[perm].set(
        jnp.arange(n, dtype=jnp.int32))
    return out_s[:n][inv]
